# baseline (speedup 1.0000x reference)
import functools

import jax
import jax.numpy as jnp
from jax import lax
from jax.experimental import pallas as pl
from jax.experimental.pallas import tpu as pltpu
import jax.experimental.pallas.tpu_sc as plsc

_TARGET = 5000
_B, _C, _V = 8, 256, 20000
_L = 16
_NCHUNK = _V // _L
_RADIX = 2048
_RBITS = 11
_NW = 32
_CPW = _C * _B // _NW
_WPB = _NW // _B
_TPAD = 5008

_OCC_BASE = 1


def _topk_body(keys_hbm, out_hbm, ka, kb, va, vb, hist, pos):
    cid = lax.axis_index("c")
    sid = lax.axis_index("s")
    wid = sid * 2 + cid

    @pl.when(wid < _B)
    def _():
        pltpu.sync_copy(keys_hbm.at[wid], ka)
        zeros = jnp.zeros((_L,), jnp.int32)
        iota = lax.iota(jnp.int32, _L)

        def zero_hist(i, _):
            hist[pl.ds(i * _L, _L)] = zeros
            return 0

        lax.fori_loop(0, _RADIX // _L, zero_hist, 0)

        def pre(i, _):
            ki = ka[pl.ds(i * _L, _L)]
            d = lax.shift_right_logical(ki, 0) & 0x7FF
            occ, lastm = plsc.scan_count(d)
            plsc.addupdate_scatter(hist, [d], occ + (1 - _OCC_BASE),
                                   mask=lastm)
            return 0

        lax.fori_loop(0, _NCHUNK, pre, 0)

        def prefix(i, carry):
            c = hist[pl.ds(i * _L, _L)]
            inc = plsc.cumsum(c)
            pos[pl.ds(i * _L, _L)] = inc - c + carry
            hist[pl.ds(i * _L, _L)] = zeros
            return carry + jnp.sum(c)

        def make_pass(shift, next_shift, ksrc, kdst, vsrc, vdst):
            lax.fori_loop(0, _RADIX // _L, prefix, jnp.int32(0))

            def perm(i, _):
                ki = ksrc[pl.ds(i * _L, _L)]
                d = lax.shift_right_logical(ki, shift) & 0x7FF
                occ, lastm = plsc.scan_count(d)
                base = plsc.load_gather(pos, [d])
                addr = base + occ - _OCC_BASE
                plsc.store_scatter(kdst, [addr], ki)
                if vsrc is None:
                    v = iota + i * _L
                else:
                    v = vsrc[pl.ds(i * _L, _L)]
                plsc.store_scatter(vdst, [addr], v)
                plsc.addupdate_scatter(pos, [d], occ + (1 - _OCC_BASE),
                                       mask=lastm)
                if next_shift is not None:
                    d2 = lax.shift_right_logical(ki, next_shift) & 0x7FF
                    occ2, lastm2 = plsc.scan_count(d2)
                    plsc.addupdate_scatter(hist, [d2],
                                           occ2 + (1 - _OCC_BASE),
                                           mask=lastm2)
                return 0

            lax.fori_loop(0, _NCHUNK, perm, 0)

        make_pass(0, _RBITS, ka, kb, None, vb)
        make_pass(_RBITS, 2 * _RBITS, kb, ka, vb, va)
        make_pass(2 * _RBITS, None, ka, kb, va, vb)

        pltpu.sync_copy(vb.at[pl.ds(0, _TARGET)],
                        out_hbm.at[pl.ds(wid * _TARGET, _TARGET)])


def _topk_idx(keys):
    mesh = plsc.VectorSubcoreMesh(core_axis_name="c", subcore_axis_name="s")
    return pl.kernel(
        _topk_body,
        out_type=jax.ShapeDtypeStruct((_B * _TARGET,), jnp.int32),
        mesh=mesh,
        compiler_params=pltpu.CompilerParams(needs_layout_passes=False),
        scratch_types=[
            pltpu.VMEM((_V,), jnp.int32),
            pltpu.VMEM((_V,), jnp.int32),
            pltpu.VMEM((_V,), jnp.int32),
            pltpu.VMEM((_V,), jnp.int32),
            pltpu.VMEM((_RADIX,), jnp.int32),
            pltpu.VMEM((_RADIX,), jnp.int32),
        ],
    )(keys)


def _gather_body(fe_hbm, idx_hbm, out_hbm, idxv, row0, row1, out0, out1,
                 sem_in0, sem_in1, sem_out0, sem_out1):
    cid = lax.axis_index("c")
    sid = lax.axis_index("s")
    wid = sid * 2 + cid
    b = wid // _WPB
    c0 = (wid % _WPB) * _CPW

    pltpu.sync_copy(idx_hbm.at[pl.ds(b * _TARGET, _TARGET)],
                    idxv.at[pl.ds(0, _TARGET)])
    iota = lax.iota(jnp.int32, _L)
    tail = idxv[pl.ds(_TPAD - _L, _L)]
    idxv[pl.ds(_TPAD - _L, _L)] = jnp.where(iota < _L - (_TPAD - _TARGET),
                                            tail, 0)

    rows = (row0, row1)
    outs = (out0, out1)
    sin = (sem_in0, sem_in1)
    sout = (sem_out0, sem_out1)

    def gather_row(row, out):
        def g(j, _):
            ii = idxv[pl.ds(j * _L, _L)]
            out[pl.ds(j * _L, _L)] = plsc.load_gather(row, [ii])
            return 0
        lax.fori_loop(0, _TPAD // _L, g, 0)

    pltpu.async_copy(fe_hbm.at[b, c0], row0, sem_in0)
    pltpu.async_copy(fe_hbm.at[b, c0 + 1], row1, sem_in1)

    rbase = (b * _C + c0) * _TARGET

    def step(c, slot):
        row, out = rows[slot], outs[slot]
        pltpu.make_async_copy(fe_hbm.at[b, c0], row, sin[slot]).wait()
        @pl.when(c >= 2)
        def _():
            pltpu.make_async_copy(out.at[pl.ds(0, _TARGET)],
                                  out_hbm.at[pl.ds(rbase, _TARGET)],
                                  sout[slot]).wait()
        gather_row(row, out)
        pltpu.async_copy(out.at[pl.ds(0, _TARGET)],
                         out_hbm.at[pl.ds(rbase + c * _TARGET, _TARGET)],
                         sout[slot])
        @pl.when(c + 2 < _CPW)
        def _():
            pltpu.async_copy(fe_hbm.at[b, c0 + c + 2], row, sin[slot])

    def loop(cc, _):
        c = cc * 2
        step(c, 0)
        step(c + 1, 1)
        return 0

    lax.fori_loop(0, _CPW // 2, loop, 0)
    pltpu.make_async_copy(out0.at[pl.ds(0, _TARGET)],
                          out_hbm.at[pl.ds(rbase, _TARGET)], sem_out0).wait()
    pltpu.make_async_copy(out1.at[pl.ds(0, _TARGET)],
                          out_hbm.at[pl.ds(rbase, _TARGET)], sem_out1).wait()


def _gather(fe, idx):
    mesh = plsc.VectorSubcoreMesh(core_axis_name="c", subcore_axis_name="s")
    return pl.kernel(
        _gather_body,
        out_type=jax.ShapeDtypeStruct((_B * _C * _TARGET,), jnp.float32),
        mesh=mesh,
        compiler_params=pltpu.CompilerParams(needs_layout_passes=False),
        scratch_types=[
            pltpu.VMEM((_TPAD,), jnp.int32),
            pltpu.VMEM((_V,), jnp.float32),
            pltpu.VMEM((_V,), jnp.float32),
            pltpu.VMEM((_TPAD,), jnp.float32),
            pltpu.VMEM((_TPAD,), jnp.float32),
            pltpu.SemaphoreType.DMA,
            pltpu.SemaphoreType.DMA,
            pltpu.SemaphoreType.DMA,
            pltpu.SemaphoreType.DMA,
        ],
    )(fe, idx)


def kernel(fe):
    scores = jnp.sum(fe * fe, axis=1)
    keys = lax.bitcast_convert_type(
        ~lax.bitcast_convert_type(scores, jnp.uint32), jnp.int32)
    idx = _topk_idx(keys)
    return _gather(fe, idx).reshape(_B, _C, _TARGET)


# speedup vs baseline: 1.1258x; 1.1258x over previous
import functools

import jax
import jax.numpy as jnp
from jax import lax
from jax.experimental import pallas as pl
from jax.experimental.pallas import tpu as pltpu
import jax.experimental.pallas.tpu_sc as plsc

_TARGET = 5000
_B, _C, _V = 8, 256, 20000
_L = 16
_NCHUNK = _V // _L
_RADIX = 2048
_RBITS = 11
_NW = 32
_CPW = _C * _B // _NW
_WPB = _NW // _B
_TPAD = 5008

_OCC_BASE = 1


def _topk_body(keys_hbm, out_hbm, ka, kb, va, vb, hist, pos):
    cid = lax.axis_index("c")
    sid = lax.axis_index("s")
    wid = sid * 2 + cid

    @pl.when(wid < _B)
    def _():
        pltpu.sync_copy(keys_hbm.at[wid], ka)
        zeros = jnp.zeros((_L,), jnp.int32)
        iota = lax.iota(jnp.int32, _L)

        def zero_hist(i, _):
            hist[pl.ds(i * _L, _L)] = zeros
            return 0

        lax.fori_loop(0, _RADIX // _L, zero_hist, 0, unroll=8)

        def pre(i, _):
            ki = ka[pl.ds(i * _L, _L)]
            d = lax.shift_right_logical(ki, 0) & 0x7FF
            occ, lastm = plsc.scan_count(d)
            plsc.addupdate_scatter(hist, [d], occ + (1 - _OCC_BASE),
                                   mask=lastm)
            return 0

        lax.fori_loop(0, _NCHUNK, pre, 0, unroll=4)

        def prefix(i, carry):
            c = hist[pl.ds(i * _L, _L)]
            inc = plsc.cumsum(c)
            pos[pl.ds(i * _L, _L)] = inc - c + carry
            hist[pl.ds(i * _L, _L)] = zeros
            return carry + jnp.sum(c)

        def make_pass(shift, next_shift, ksrc, kdst, vsrc, vdst):
            lax.fori_loop(0, _RADIX // _L, prefix, jnp.int32(0))

            def perm(i, _):
                ki = ksrc[pl.ds(i * _L, _L)]
                d = lax.shift_right_logical(ki, shift) & 0x7FF
                occ, lastm = plsc.scan_count(d)
                base = plsc.load_gather(pos, [d])
                addr = base + occ - _OCC_BASE
                plsc.store_scatter(kdst, [addr], ki)
                if vsrc is None:
                    v = iota + i * _L
                else:
                    v = vsrc[pl.ds(i * _L, _L)]
                plsc.store_scatter(vdst, [addr], v)
                plsc.addupdate_scatter(pos, [d], occ + (1 - _OCC_BASE),
                                       mask=lastm)
                if next_shift is not None:
                    d2 = lax.shift_right_logical(ki, next_shift) & 0x7FF
                    occ2, lastm2 = plsc.scan_count(d2)
                    plsc.addupdate_scatter(hist, [d2],
                                           occ2 + (1 - _OCC_BASE),
                                           mask=lastm2)
                return 0

            lax.fori_loop(0, _NCHUNK, perm, 0, unroll=2)

        make_pass(0, _RBITS, ka, kb, None, vb)
        make_pass(_RBITS, 2 * _RBITS, kb, ka, vb, va)
        make_pass(2 * _RBITS, None, ka, kb, va, vb)

        pltpu.sync_copy(vb.at[pl.ds(0, _TARGET)],
                        out_hbm.at[pl.ds(wid * _TARGET, _TARGET)])


def _topk_idx(keys):
    mesh = plsc.VectorSubcoreMesh(core_axis_name="c", subcore_axis_name="s")
    return pl.kernel(
        _topk_body,
        out_type=jax.ShapeDtypeStruct((_B * _TARGET,), jnp.int32),
        mesh=mesh,
        compiler_params=pltpu.CompilerParams(needs_layout_passes=False),
        scratch_types=[
            pltpu.VMEM((_V,), jnp.int32),
            pltpu.VMEM((_V,), jnp.int32),
            pltpu.VMEM((_V,), jnp.int32),
            pltpu.VMEM((_V,), jnp.int32),
            pltpu.VMEM((_RADIX,), jnp.int32),
            pltpu.VMEM((_RADIX,), jnp.int32),
        ],
    )(keys)


def _gather_body(fe_hbm, idx_hbm, out_hbm, idxv, row0, row1, out0, out1,
                 sem_in0, sem_in1, sem_out0, sem_out1):
    cid = lax.axis_index("c")
    sid = lax.axis_index("s")
    wid = sid * 2 + cid
    b = wid // _WPB
    c0 = (wid % _WPB) * _CPW

    pltpu.sync_copy(idx_hbm.at[pl.ds(b * _TARGET, _TARGET)],
                    idxv.at[pl.ds(0, _TARGET)])
    iota = lax.iota(jnp.int32, _L)
    tail = idxv[pl.ds(_TPAD - _L, _L)]
    idxv[pl.ds(_TPAD - _L, _L)] = jnp.where(iota < _L - (_TPAD - _TARGET),
                                            tail, 0)

    rows = (row0, row1)
    outs = (out0, out1)
    sin = (sem_in0, sem_in1)
    sout = (sem_out0, sem_out1)

    def gather_row(row, out):
        @plsc.parallel_loop(0, _TPAD // _L, unroll=8)
        def _(j):
            ii = idxv[pl.ds(j * _L, _L)]
            out[pl.ds(j * _L, _L)] = plsc.load_gather(row, [ii])

    pltpu.async_copy(fe_hbm.at[b, c0], row0, sem_in0)
    pltpu.async_copy(fe_hbm.at[b, c0 + 1], row1, sem_in1)

    rbase = (b * _C + c0) * _TARGET

    def step(c, slot):
        row, out = rows[slot], outs[slot]
        pltpu.make_async_copy(fe_hbm.at[b, c0], row, sin[slot]).wait()
        @pl.when(c >= 2)
        def _():
            pltpu.make_async_copy(out.at[pl.ds(0, _TARGET)],
                                  out_hbm.at[pl.ds(rbase, _TARGET)],
                                  sout[slot]).wait()
        gather_row(row, out)
        pltpu.async_copy(out.at[pl.ds(0, _TARGET)],
                         out_hbm.at[pl.ds(rbase + c * _TARGET, _TARGET)],
                         sout[slot])
        @pl.when(c + 2 < _CPW)
        def _():
            pltpu.async_copy(fe_hbm.at[b, c0 + c + 2], row, sin[slot])

    def loop(cc, _):
        c = cc * 2
        step(c, 0)
        step(c + 1, 1)
        return 0

    lax.fori_loop(0, _CPW // 2, loop, 0)
    pltpu.make_async_copy(out0.at[pl.ds(0, _TARGET)],
                          out_hbm.at[pl.ds(rbase, _TARGET)], sem_out0).wait()
    pltpu.make_async_copy(out1.at[pl.ds(0, _TARGET)],
                          out_hbm.at[pl.ds(rbase, _TARGET)], sem_out1).wait()


def _gather(fe, idx):
    mesh = plsc.VectorSubcoreMesh(core_axis_name="c", subcore_axis_name="s")
    return pl.kernel(
        _gather_body,
        out_type=jax.ShapeDtypeStruct((_B * _C * _TARGET,), jnp.float32),
        mesh=mesh,
        compiler_params=pltpu.CompilerParams(needs_layout_passes=False),
        scratch_types=[
            pltpu.VMEM((_TPAD,), jnp.int32),
            pltpu.VMEM((_V,), jnp.float32),
            pltpu.VMEM((_V,), jnp.float32),
            pltpu.VMEM((_TPAD,), jnp.float32),
            pltpu.VMEM((_TPAD,), jnp.float32),
            pltpu.SemaphoreType.DMA,
            pltpu.SemaphoreType.DMA,
            pltpu.SemaphoreType.DMA,
            pltpu.SemaphoreType.DMA,
        ],
    )(fe, idx)


def kernel(fe):
    scores = jnp.sum(fe * fe, axis=1)
    keys = lax.bitcast_convert_type(
        ~lax.bitcast_convert_type(scores, jnp.uint32), jnp.int32)
    idx = _topk_idx(keys)
    return _gather(fe, idx).reshape(_B, _C, _TARGET)


# speedup vs baseline: 1.2151x; 1.0793x over previous
import functools

import jax
import jax.numpy as jnp
from jax import lax
from jax.experimental import pallas as pl
from jax.experimental.pallas import tpu as pltpu
import jax.experimental.pallas.tpu_sc as plsc

_TARGET = 5000
_B, _C, _V = 8, 256, 20000
_L = 16
_NCHUNK = _V // _L
_RADIX = 2048
_RBITS = 11
_NW = 32
_CPW = _C * _B // _NW
_WPB = _NW // _B
_TPAD = 5008

_OCC_BASE = 1


def _topk_body(keys_hbm, out_hbm, ka, kb, va, vb, hist, pos):
    cid = lax.axis_index("c")
    sid = lax.axis_index("s")
    wid = sid * 2 + cid

    @pl.when(wid < _B)
    def _():
        pltpu.sync_copy(keys_hbm.at[wid], ka)
        zeros = jnp.zeros((_L,), jnp.int32)
        iota = lax.iota(jnp.int32, _L)

        def zero_hist(i, _):
            hist[pl.ds(i * _L, _L)] = zeros
            return 0

        lax.fori_loop(0, _RADIX // _L, zero_hist, 0, unroll=8)

        def pre(i, _):
            ki = ka[pl.ds(i * _L, _L)]
            d = lax.shift_right_logical(ki, 0) & 0x7FF
            occ, lastm = plsc.scan_count(d)
            plsc.addupdate_scatter(hist, [d], occ + (1 - _OCC_BASE),
                                   mask=lastm)
            return 0

        lax.fori_loop(0, _NCHUNK, pre, 0, unroll=4)

        def prefix(i, carry):
            c = hist[pl.ds(i * _L, _L)]
            inc = plsc.cumsum(c)
            pos[pl.ds(i * _L, _L)] = inc - c + carry
            hist[pl.ds(i * _L, _L)] = zeros
            return carry + jnp.sum(c)

        def make_pass(shift, next_shift, ksrc, kdst, vsrc, vdst):
            lax.fori_loop(0, _RADIX // _L, prefix, jnp.int32(0))

            def perm(i, _):
                ki = ksrc[pl.ds(i * _L, _L)]
                d = lax.shift_right_logical(ki, shift) & 0x7FF
                occ, lastm = plsc.scan_count(d)
                base = plsc.load_gather(pos, [d])
                addr = base + occ - _OCC_BASE
                plsc.store_scatter(kdst, [addr], ki)
                if vsrc is None:
                    v = iota + i * _L
                else:
                    v = vsrc[pl.ds(i * _L, _L)]
                plsc.store_scatter(vdst, [addr], v)
                plsc.addupdate_scatter(pos, [d], occ + (1 - _OCC_BASE),
                                       mask=lastm)
                if next_shift is not None:
                    d2 = lax.shift_right_logical(ki, next_shift) & 0x7FF
                    occ2, lastm2 = plsc.scan_count(d2)
                    plsc.addupdate_scatter(hist, [d2],
                                           occ2 + (1 - _OCC_BASE),
                                           mask=lastm2)
                return 0

            lax.fori_loop(0, _NCHUNK, perm, 0, unroll=2)

        make_pass(0, _RBITS, ka, kb, None, vb)
        make_pass(_RBITS, 2 * _RBITS, kb, ka, vb, va)
        make_pass(2 * _RBITS, None, ka, kb, va, vb)

        pltpu.sync_copy(vb.at[pl.ds(0, _TARGET)],
                        out_hbm.at[pl.ds(wid * _TARGET, _TARGET)])


def _topk_idx(keys):
    mesh = plsc.VectorSubcoreMesh(core_axis_name="c", subcore_axis_name="s")
    return pl.kernel(
        _topk_body,
        out_type=jax.ShapeDtypeStruct((_B * _TARGET,), jnp.int32),
        mesh=mesh,
        compiler_params=pltpu.CompilerParams(needs_layout_passes=False),
        scratch_types=[
            pltpu.VMEM((_V,), jnp.int32),
            pltpu.VMEM((_V,), jnp.int32),
            pltpu.VMEM((_V,), jnp.int32),
            pltpu.VMEM((_V,), jnp.int32),
            pltpu.VMEM((_RADIX,), jnp.int32),
            pltpu.VMEM((_RADIX,), jnp.int32),
        ],
    )(keys)


_GPW = _CPW // 8


def _gather_body(fe_hbm, idx_hbm, out_hbm, idxv, row0, row1, slab0, slab1,
                 sem_in0, sem_in1, sem_out0, sem_out1):
    cid = lax.axis_index("c")
    sid = lax.axis_index("s")
    wid = sid * 2 + cid
    b = wid // _WPB
    c0 = (wid % _WPB) * _CPW

    pltpu.sync_copy(idx_hbm.at[pl.ds(b * _TARGET, _TARGET)],
                    idxv.at[pl.ds(0, _TARGET)])
    iota = lax.iota(jnp.int32, _L)
    tail = idxv[pl.ds(_TPAD - _L, _L)]
    idxv[pl.ds(_TPAD - _L, _L)] = jnp.where(iota < _L - (_TPAD - _TARGET),
                                            tail, 0)

    rows = (row0, row1)
    slabs = (slab0, slab1)
    sin = (sem_in0, sem_in1)
    sout = (sem_out0, sem_out1)

    nfull = _TARGET // _L

    def gather_row(row, slab, cr):
        @plsc.parallel_loop(0, nfull, unroll=8)
        def _(j):
            ii = idxv[pl.ds(j * _L, _L)]
            slab[cr, pl.ds(j * _L, _L)] = plsc.load_gather(row, [ii])

        ii = idxv[pl.ds(nfull * _L, _L)]
        vals = plsc.load_gather(row, [ii])
        plsc.store_scatter(slab, [jnp.full((_L,), cr, jnp.int32),
                                  iota + nfull * _L], vals,
                           mask=iota < _TARGET - nfull * _L)

    pltpu.async_copy(fe_hbm.at[b, c0], row0, sem_in0)
    pltpu.async_copy(fe_hbm.at[b, c0 + 1], row1, sem_in1)

    for g in range(_GPW):
        gslot = g % 2
        slab = slabs[gslot]
        if g >= 2:
            pltpu.make_async_copy(slab, out_hbm.at[b, pl.ds(0, 8)],
                                  sout[gslot]).wait()
        for cr in range(8):
            r = g * 8 + cr
            rslot = r % 2
            row = rows[rslot]
            pltpu.make_async_copy(fe_hbm.at[b, c0], row, sin[rslot]).wait()
            gather_row(row, slab, cr)
            if r + 2 < _CPW:
                pltpu.async_copy(fe_hbm.at[b, c0 + r + 2], row, sin[rslot])
        pltpu.async_copy(slab, out_hbm.at[b, pl.ds(c0 + 8 * g, 8)],
                         sout[gslot])

    pltpu.make_async_copy(slab0, out_hbm.at[b, pl.ds(0, 8)],
                          sem_out0).wait()
    pltpu.make_async_copy(slab1, out_hbm.at[b, pl.ds(0, 8)],
                          sem_out1).wait()


def _gather(fe, idx):
    mesh = plsc.VectorSubcoreMesh(core_axis_name="c", subcore_axis_name="s")
    return pl.kernel(
        _gather_body,
        out_type=jax.ShapeDtypeStruct((_B, _C, _TARGET), jnp.float32),
        mesh=mesh,
        compiler_params=pltpu.CompilerParams(needs_layout_passes=False),
        scratch_types=[
            pltpu.VMEM((_TPAD,), jnp.int32),
            pltpu.VMEM((_V,), jnp.float32),
            pltpu.VMEM((_V,), jnp.float32),
            pltpu.VMEM((8, _TARGET), jnp.float32),
            pltpu.VMEM((8, _TARGET), jnp.float32),
            pltpu.SemaphoreType.DMA,
            pltpu.SemaphoreType.DMA,
            pltpu.SemaphoreType.DMA,
            pltpu.SemaphoreType.DMA,
        ],
    )(fe, idx)


def kernel(fe):
    scores = jnp.sum(fe * fe, axis=1)
    keys = lax.bitcast_convert_type(
        ~lax.bitcast_convert_type(scores, jnp.uint32), jnp.int32)
    idx = _topk_idx(keys)
    return _gather(fe, idx)


# speedup vs baseline: 1.2168x; 1.0014x over previous
import functools

import jax
import jax.numpy as jnp
from jax import lax
from jax.experimental import pallas as pl
from jax.experimental.pallas import tpu as pltpu
import jax.experimental.pallas.tpu_sc as plsc

_TARGET = 5000
_B, _C, _V = 8, 256, 20000
_L = 16
_NCHUNK = _V // _L
_RADIX = 8192
_RBITS = 13
_DMASK = _RADIX - 1
_NW = 32
_CPW = _C * _B // _NW
_WPB = _NW // _B
_TPAD = 5008

_OCC_BASE = 1


def _topk_body(keys_hbm, out_hbm, ka, kb, va, vb, hista, histb, pos):
    cid = lax.axis_index("c")
    sid = lax.axis_index("s")
    wid = sid * 2 + cid

    @pl.when(wid < _B)
    def _():
        pltpu.sync_copy(keys_hbm.at[wid], ka)
        zeros = jnp.zeros((_L,), jnp.int32)
        iota = lax.iota(jnp.int32, _L)

        def zero(ref):
            def z(i, _):
                ref[pl.ds(i * _L, _L)] = zeros
                return 0
            lax.fori_loop(0, _RADIX // _L, z, 0, unroll=8)

        zero(hista)
        zero(histb)

        def minmax(i, c):
            ki = ka[pl.ds(i * _L, _L)]
            return (jnp.minimum(c[0], ki), jnp.maximum(c[1], ki))

        mn_v, mx_v = lax.fori_loop(
            0, _NCHUNK, minmax,
            (jnp.full((_L,), 0x7FFFFFFF, jnp.int32),
             jnp.full((_L,), -0x80000000, jnp.int32)), unroll=4)
        kmin = jnp.min(mn_v)
        need3 = (jnp.max(mx_v) - kmin) >= (1 << (2 * _RBITS))

        def pre(i, _):
            ki = ka[pl.ds(i * _L, _L)] - kmin
            ka[pl.ds(i * _L, _L)] = ki
            da = ki & _DMASK
            occa, lasta = plsc.scan_count(da)
            plsc.addupdate_scatter(hista, [da], occa, mask=lasta)
            db = lax.shift_right_logical(ki, _RBITS) & _DMASK
            occb, lastb = plsc.scan_count(db)
            plsc.addupdate_scatter(histb, [db], occb, mask=lastb)
            return 0

        lax.fori_loop(0, _NCHUNK, pre, 0, unroll=2)

        def prefix(hist):
            def p(i, carry):
                c = hist[pl.ds(i * _L, _L)]
                inc = plsc.cumsum(c)
                pos[pl.ds(i * _L, _L)] = inc - c + carry
                return carry + jnp.sum(c)
            lax.fori_loop(0, _RADIX // _L, p, jnp.int32(0), unroll=2)

        def make_pass(shift, ksrc, kdst, vsrc, vdst):
            def perm(i, _):
                ki = ksrc[pl.ds(i * _L, _L)]
                d = lax.shift_right_logical(ki, shift) & _DMASK
                occ, lastm = plsc.scan_count(d)
                base = plsc.load_gather(pos, [d])
                addr = base + occ - _OCC_BASE
                plsc.store_scatter(kdst, [addr], ki)
                if vsrc is None:
                    v = iota + i * _L
                else:
                    v = vsrc[pl.ds(i * _L, _L)]
                plsc.store_scatter(vdst, [addr], v)
                plsc.addupdate_scatter(pos, [d], occ, mask=lastm)
                return 0

            lax.fori_loop(0, _NCHUNK, perm, 0, unroll=2)

        prefix(hista)
        make_pass(0, ka, kb, None, vb)
        prefix(histb)
        make_pass(_RBITS, kb, ka, vb, va)

        @pl.when(need3)
        def _():
            zero(hista)

            def hi_hist(i, _):
                d = lax.shift_right_logical(ka[pl.ds(i * _L, _L)],
                                            2 * _RBITS) & _DMASK
                occ, lastm = plsc.scan_count(d)
                plsc.addupdate_scatter(hista, [d], occ, mask=lastm)
                return 0

            lax.fori_loop(0, _NCHUNK, hi_hist, 0, unroll=4)
            prefix(hista)
            make_pass(2 * _RBITS, ka, kb, va, vb)
            pltpu.sync_copy(vb.at[pl.ds(0, _TARGET)],
                            out_hbm.at[pl.ds(wid * _TARGET, _TARGET)])

        @pl.when(jnp.logical_not(need3))
        def _():
            pltpu.sync_copy(va.at[pl.ds(0, _TARGET)],
                            out_hbm.at[pl.ds(wid * _TARGET, _TARGET)])


def _topk_idx(keys):
    mesh = plsc.VectorSubcoreMesh(core_axis_name="c", subcore_axis_name="s")
    return pl.kernel(
        _topk_body,
        out_type=jax.ShapeDtypeStruct((_B * _TARGET,), jnp.int32),
        mesh=mesh,
        compiler_params=pltpu.CompilerParams(needs_layout_passes=False),
        scratch_types=[
            pltpu.VMEM((_V,), jnp.int32),
            pltpu.VMEM((_V,), jnp.int32),
            pltpu.VMEM((_V,), jnp.int32),
            pltpu.VMEM((_V,), jnp.int32),
            pltpu.VMEM((_RADIX,), jnp.int32),
            pltpu.VMEM((_RADIX,), jnp.int32),
            pltpu.VMEM((_RADIX,), jnp.int32),
        ],
    )(keys)


_GPW = _CPW // 8


def _gather_body(fe_hbm, idx_hbm, out_hbm, idxv, row0, row1, slab0, slab1,
                 sem_in0, sem_in1, sem_out0, sem_out1):
    cid = lax.axis_index("c")
    sid = lax.axis_index("s")
    wid = sid * 2 + cid
    b = wid // _WPB
    c0 = (wid % _WPB) * _CPW

    pltpu.sync_copy(idx_hbm.at[pl.ds(b * _TARGET, _TARGET)],
                    idxv.at[pl.ds(0, _TARGET)])
    iota = lax.iota(jnp.int32, _L)
    tail = idxv[pl.ds(_TPAD - _L, _L)]
    idxv[pl.ds(_TPAD - _L, _L)] = jnp.where(iota < _L - (_TPAD - _TARGET),
                                            tail, 0)

    rows = (row0, row1)
    slabs = (slab0, slab1)
    sin = (sem_in0, sem_in1)
    sout = (sem_out0, sem_out1)

    nfull = _TARGET // _L

    def gather_row(row, slab, cr):
        @plsc.parallel_loop(0, nfull, unroll=8)
        def _(j):
            ii = idxv[pl.ds(j * _L, _L)]
            slab[cr, pl.ds(j * _L, _L)] = plsc.load_gather(row, [ii])

        ii = idxv[pl.ds(nfull * _L, _L)]
        vals = plsc.load_gather(row, [ii])
        plsc.store_scatter(slab, [jnp.full((_L,), cr, jnp.int32),
                                  iota + nfull * _L], vals,
                           mask=iota < _TARGET - nfull * _L)

    pltpu.async_copy(fe_hbm.at[b, c0], row0, sem_in0)
    pltpu.async_copy(fe_hbm.at[b, c0 + 1], row1, sem_in1)

    for g in range(_GPW):
        gslot = g % 2
        slab = slabs[gslot]
        if g >= 2:
            pltpu.make_async_copy(slab, out_hbm.at[b, pl.ds(0, 8)],
                                  sout[gslot]).wait()
        for cr in range(8):
            r = g * 8 + cr
            rslot = r % 2
            row = rows[rslot]
            pltpu.make_async_copy(fe_hbm.at[b, c0], row, sin[rslot]).wait()
            gather_row(row, slab, cr)
            if r + 2 < _CPW:
                pltpu.async_copy(fe_hbm.at[b, c0 + r + 2], row, sin[rslot])
        pltpu.async_copy(slab, out_hbm.at[b, pl.ds(c0 + 8 * g, 8)],
                         sout[gslot])

    pltpu.make_async_copy(slab0, out_hbm.at[b, pl.ds(0, 8)],
                          sem_out0).wait()
    pltpu.make_async_copy(slab1, out_hbm.at[b, pl.ds(0, 8)],
                          sem_out1).wait()


def _gather(fe, idx):
    mesh = plsc.VectorSubcoreMesh(core_axis_name="c", subcore_axis_name="s")
    return pl.kernel(
        _gather_body,
        out_type=jax.ShapeDtypeStruct((_B, _C, _TARGET), jnp.float32),
        mesh=mesh,
        compiler_params=pltpu.CompilerParams(needs_layout_passes=False),
        scratch_types=[
            pltpu.VMEM((_TPAD,), jnp.int32),
            pltpu.VMEM((_V,), jnp.float32),
            pltpu.VMEM((_V,), jnp.float32),
            pltpu.VMEM((8, _TARGET), jnp.float32),
            pltpu.VMEM((8, _TARGET), jnp.float32),
            pltpu.SemaphoreType.DMA,
            pltpu.SemaphoreType.DMA,
            pltpu.SemaphoreType.DMA,
            pltpu.SemaphoreType.DMA,
        ],
    )(fe, idx)


def kernel(fe):
    scores = jnp.sum(fe * fe, axis=1)
    keys = lax.bitcast_convert_type(
        ~lax.bitcast_convert_type(scores, jnp.uint32), jnp.int32)
    idx = _topk_idx(keys)
    return _gather(fe, idx)
